# baseline (speedup 1.0000x reference)
import functools

import jax
import jax.numpy as jnp
from jax.experimental import pallas as pl
from jax.experimental.pallas import tpu as pltpu

DEADE = float(-(1 << 22))
MANT_MASK = 0x007FFFFF
ONE_BITS = 0x3F800000
LN2HI = 0.69314575195
LN2LO = 1.42860677e-06
MAGIC = 12582912.0


def _gather_kernel(lp_ref, cls_ref, w_ref):
    x = lp_ref[0]
    m = jnp.max(x, axis=1, keepdims=True)
    e = jnp.exp(x - m)
    z = jnp.sum(e, axis=1, keepdims=True)
    C = x.shape[1]
    cls = cls_ref[0]
    cidx = jax.lax.broadcasted_iota(jnp.int32, (C, 128), 0)
    oh = (cidx == cls).astype(jnp.bfloat16)
    g = jnp.dot(e.astype(jnp.bfloat16), oh,
                preferred_element_type=jnp.float32)
    w_ref[0] = g * (1.0 / z)


def _decomp(p):
    bits = jax.lax.bitcast_convert_type(p, jnp.int32)
    e = jax.lax.shift_right_logical(bits, 23).astype(jnp.float32) - 127.0
    m = jax.lax.bitcast_convert_type(
        jax.lax.bitwise_or(jax.lax.bitwise_and(bits, MANT_MASK), ONE_BITS),
        jnp.float32)
    return m, e


def _scale(diff):
    d = diff.astype(jnp.int32)
    return jax.lax.bitcast_convert_type(
        jax.lax.shift_left(jnp.maximum(d + 127, 0), 23), jnp.float32)


def _renorm(m):
    bits = jax.lax.bitcast_convert_type(m, jnp.int32)
    de = jax.lax.shift_right_logical(bits, 23).astype(jnp.float32) - 127.0
    mn = jax.lax.bitcast_convert_type(
        jax.lax.bitwise_or(jax.lax.bitwise_and(bits, MANT_MASK), ONE_BITS),
        jnp.float32)
    return mn, de


def _snap(x):
    return (x + MAGIC) - MAGIC


def _dp_kernel(w_ref, skip_ref, len_ref, selb_ref, sela_ref, out_ref,
               mA_r, eA_r, m0_r, e0_r, *, tb):
    i = pl.program_id(0)
    nt = pl.num_programs(0)
    b = skip_ref.shape[0]
    lane = jax.lax.broadcasted_iota(jnp.int32, (b, 128), 1)
    row = jax.lax.broadcasted_iota(jnp.int32, (128, 128), 0)
    col = jax.lax.broadcasted_iota(jnp.int32, (128, 128), 1)
    d1 = (col == row + 1).astype(jnp.float32)
    d2 = (col == row + 2).astype(jnp.float32)
    dm1 = (row == col + 1).astype(jnp.float32)

    @pl.when(i == 0)
    def _init():
        p0 = w_ref[0]
        mi, ei = _decomp(p0)
        mA_r[...] = jnp.where(lane == 0, mi, 1.0)
        eA_r[...] = jnp.where(lane == 0, ei, DEADE)
        pb0 = jnp.dot(p0, dm1, preferred_element_type=jnp.float32)
        m0i, e0i = _decomp(pb0)
        m0_r[...] = m0i
        e0_r[...] = e0i

    skipm = skip_ref[...] > 0
    leni = len_ref[...]
    mA = mA_r[...]
    eA = eA_r[...]
    m0 = m0_r[...]
    e0 = e0_r[...]
    for tt in range(tb):
        t = i * tb + tt
        p = w_ref[tt]
        pdn = jnp.dot(p, dm1, preferred_element_type=jnp.float32)
        m1 = jnp.dot(mA, d1, preferred_element_type=jnp.float32)
        e1 = jnp.dot(eA, d1, preferred_element_type=jnp.float32)
        m2 = jnp.dot(mA, d2, preferred_element_type=jnp.float32)
        e2 = jnp.dot(eA, d2, preferred_element_type=jnp.float32)
        m1 = jnp.where(lane == 0, m0, m1)
        e1 = jnp.where(lane == 0, e0, e1)
        e2 = jnp.where(skipm, e2, DEADE)
        E = jnp.maximum(jnp.maximum(eA, e1), e2)
        msum = (mA * _scale(eA - E) + m1 * _scale(e1 - E)
                + m2 * _scale(e2 - E)) * p
        mN, de = _renorm(msum)
        eN = _snap(E + de)
        m0m = m0 * pdn
        m0n, de0 = _renorm(m0m)
        e0n = _snap(e0 + de0)
        act = (t < leni) & (t > 0)
        mA = jnp.where(act, mN, mA)
        eA = jnp.where(act, eN, eA)
        m0 = jnp.where(act, m0n, m0)
        e0 = jnp.where(act, e0n, e0)
    mA_r[...] = mA
    eA_r[...] = eA
    m0_r[...] = m0
    e0_r[...] = e0

    @pl.when(i == nt - 1)
    def _fin():
        selb = selb_ref[...] > 0
        sela = sela_ref[...] > 0
        mb = jnp.max(jnp.where(selb, mA, 0.0), axis=1, keepdims=True)
        ebx = jnp.max(jnp.where(selb, eA, DEADE), axis=1, keepdims=True)
        ma = jnp.max(jnp.where(sela, mA, 0.0), axis=1, keepdims=True)
        eax = jnp.max(jnp.where(sela, eA, DEADE), axis=1, keepdims=True)
        E2 = jnp.maximum(ebx, eax)
        v = mb * _scale(ebx - E2) + ma * _scale(eax - E2)
        e2f = E2.astype(jnp.float32)
        loss = -(jnp.log(v) + e2f * LN2HI + e2f * LN2LO)
        out_ref[...] = jnp.broadcast_to(loss, (b, 128))


@jax.jit
def kernel(log_probs, targets, input_lengths, target_lengths):
    B, T, C = log_probs.shape
    L = targets.shape[1]
    targets = targets.astype(jnp.int32)
    input_lengths = input_lengths.astype(jnp.int32)
    target_lengths = target_lengths.astype(jnp.int32)

    lane = jnp.arange(128, dtype=jnp.int32)[None, :]
    lab_of_lane = jnp.clip(lane // 2, 0, L - 1)
    tgt_at_lane = jnp.take_along_axis(
        targets, jnp.broadcast_to(lab_of_lane, (B, 128)), axis=1)
    cls = jnp.where(lane % 2 == 0, tgt_at_lane, 0)
    cls = cls[:, None, :]
    prev_at_lane = jnp.take_along_axis(
        targets, jnp.broadcast_to(jnp.clip(lane // 2 - 1, 0, L - 1), (B, 128)),
        axis=1)
    skip = (lane % 2 == 0) & (lane >= 2) & (tgt_at_lane != prev_at_lane)
    skipf = skip.astype(jnp.float32)
    lenb = jnp.broadcast_to(input_lengths[:, None], (B, 128))
    selb = (lane == 2 * target_lengths[:, None] - 1).astype(jnp.float32)
    sela = (lane == 2 * target_lengths[:, None] - 2).astype(jnp.float32)

    w = pl.pallas_call(
        _gather_kernel,
        grid=(B,),
        in_specs=[
            pl.BlockSpec((1, T, C), lambda i: (i, 0, 0)),
            pl.BlockSpec((1, 1, 128), lambda i: (i, 0, 0)),
        ],
        out_specs=pl.BlockSpec((1, T, 128), lambda i: (i, 0, 0)),
        out_shape=jax.ShapeDtypeStruct((B, T, 128), jnp.float32),
        compiler_params=pltpu.CompilerParams(
            dimension_semantics=("arbitrary",)),
    )(log_probs, cls)

    wt = jnp.transpose(w, (1, 0, 2))

    TB = 64
    NT = T // TB
    out = pl.pallas_call(
        functools.partial(_dp_kernel, tb=TB),
        grid=(NT,),
        in_specs=[
            pl.BlockSpec((TB, B, 128), lambda i: (i, 0, 0)),
            pl.BlockSpec((B, 128), lambda i: (0, 0)),
            pl.BlockSpec((B, 128), lambda i: (0, 0)),
            pl.BlockSpec((B, 128), lambda i: (0, 0)),
            pl.BlockSpec((B, 128), lambda i: (0, 0)),
        ],
        out_specs=pl.BlockSpec((B, 128), lambda i: (0, 0)),
        out_shape=jax.ShapeDtypeStruct((B, 128), jnp.float32),
        scratch_shapes=[
            pltpu.VMEM((B, 128), jnp.float32),
            pltpu.VMEM((B, 128), jnp.float32),
            pltpu.VMEM((B, 128), jnp.float32),
            pltpu.VMEM((B, 128), jnp.float32),
        ],
        compiler_params=pltpu.CompilerParams(
            dimension_semantics=("arbitrary",)),
    )(wt, skipf, lenb, selb, sela)

    return out[:, 0]


# speedup vs baseline: 1.1741x; 1.1741x over previous
import functools

import jax
import jax.numpy as jnp
from jax import lax
from jax.experimental import pallas as pl
from jax.experimental.pallas import tpu as pltpu
from jax.experimental.pallas import tpu_sc as plsc

DEADE = -(1 << 28)
MANT_MASK = 0x007FFFFF
ONE_BITS = 0x3F800000
LN2HI = 0.69314575195
LN2LO = 1.42860677e-06

NCHUNK = 9
BUFLEN = 160


def _gather_kernel(lp_ref, cls_ref, w_ref):
    x = lp_ref[0]
    m = jnp.max(x, axis=1, keepdims=True)
    e = jnp.exp(x - m)
    z = jnp.sum(e, axis=1, keepdims=True)
    C = x.shape[1]
    cls = cls_ref[0]
    cidx = jax.lax.broadcasted_iota(jnp.int32, (C, 128), 0)
    oh = (cidx == cls).astype(jnp.bfloat16)
    g = jnp.dot(e.astype(jnp.bfloat16), oh,
                preferred_element_type=jnp.float32)
    w_ref[0] = g * (1.0 / z)


def _vdecomp(p):
    bits = lax.bitcast_convert_type(p, jnp.int32)
    e = lax.shift_right_logical(bits, 23) - 127
    m = lax.bitcast_convert_type(
        lax.bitwise_or(lax.bitwise_and(bits, MANT_MASK), ONE_BITS),
        jnp.float32)
    return m, e


def _vscale(d):
    return lax.bitcast_convert_type(
        lax.shift_left(jnp.maximum(d + 127, 0), 23), jnp.float32)


def _sc_dp_build(B, T):
    mesh = plsc.VectorSubcoreMesh(core_axis_name="c", subcore_axis_name="s")

    @functools.partial(
        pl.kernel,
        mesh=mesh,
        out_type=[
            jax.ShapeDtypeStruct((B, BUFLEN), jnp.float32),
            jax.ShapeDtypeStruct((B, BUFLEN), jnp.int32),
        ],
        scratch_types=[
            pltpu.VMEM((T * 128,), jnp.float32),
            pltpu.VMEM((BUFLEN,), jnp.float32),
            pltpu.VMEM((16,), jnp.int32),
            pltpu.VMEM((BUFLEN,), jnp.float32),
            pltpu.VMEM((BUFLEN,), jnp.int32),
            pltpu.VMEM((BUFLEN,), jnp.float32),
            pltpu.VMEM((BUFLEN,), jnp.int32),
        ],
    )
    def sc_dp(w_hbm, skip_hbm, len_hbm, m_out, e_out,
              w_v, skip_v, len_v, bmA, beA, bmB, beB):
        wid = lax.axis_index("s") * 2 + lax.axis_index("c")
        pltpu.sync_copy(w_hbm.at[wid], w_v)
        pltpu.sync_copy(skip_hbm.at[wid], skip_v)
        pltpu.sync_copy(len_hbm.at[wid], len_v)

        ones = jnp.full((16,), 1.0, jnp.float32)
        deade = jnp.full((16,), DEADE, jnp.int32)
        for off in range(0, BUFLEN, 16):
            bmA[pl.ds(off, 16)] = ones
            beA[pl.ds(off, 16)] = deade
            bmB[pl.ds(off, 16)] = ones
            beB[pl.ds(off, 16)] = deade
        p0 = w_v[pl.ds(0, 16)]
        m_i, e_i = _vdecomp(p0)
        lane16 = lax.broadcasted_iota(jnp.int32, (16,), 0)
        bmA[pl.ds(2, 16)] = jnp.where(lane16 < 2, m_i, 1.0)
        beA[pl.ds(2, 16)] = jnp.where(lane16 < 2, e_i, DEADE)

        lenv = len_v[...]
        skv = [skip_v[pl.ds(16 * c, 16)] for c in range(NCHUNK)]

        def step(t, src_m, src_e, dst_m, dst_e):
            act = jnp.full((16,), t, jnp.int32) < lenv
            for c in range(NCHUNK):
                base = 16 * c
                prow = min(base, 127 - 15)
                pch = w_v[pl.ds(t * 128 + prow, 16)]
                mS = src_m[pl.ds(base + 2, 16)]
                eS = src_e[pl.ds(base + 2, 16)]
                m1 = src_m[pl.ds(base + 1, 16)]
                e1 = src_e[pl.ds(base + 1, 16)]
                m2 = src_m[pl.ds(base, 16)]
                e2 = src_e[pl.ds(base, 16)]
                e2 = jnp.where(skv[c] > 0, e2, DEADE)
                E = jnp.maximum(jnp.maximum(eS, e1), e2)
                msum = (mS * _vscale(eS - E) + m1 * _vscale(e1 - E)
                        + m2 * _vscale(e2 - E)) * pch
                bits = lax.bitcast_convert_type(msum, jnp.int32)
                eb = lax.shift_right_logical(bits, 23)
                mN = lax.bitcast_convert_type(
                    lax.bitwise_or(lax.bitwise_and(bits, MANT_MASK),
                                   ONE_BITS), jnp.float32)
                eN = E + (eb - 127)
                dst_m[pl.ds(base + 2, 16)] = jnp.where(act, mN, mS)
                dst_e[pl.ds(base + 2, 16)] = jnp.where(act, eN, eS)

        def pair(k, carry):
            step(1 + 2 * k, bmA, beA, bmB, beB)
            step(2 + 2 * k, bmB, beB, bmA, beA)
            return carry

        lax.fori_loop(0, (T - 2) // 2, pair, 0)
        step(T - 1, bmA, beA, bmB, beB)

        pltpu.sync_copy(bmB, m_out.at[wid])
        pltpu.sync_copy(beB, e_out.at[wid])

    return sc_dp


@jax.jit
def kernel(log_probs, targets, input_lengths, target_lengths):
    B, T, C = log_probs.shape
    L = targets.shape[1]
    targets = targets.astype(jnp.int32)
    input_lengths = input_lengths.astype(jnp.int32)
    target_lengths = target_lengths.astype(jnp.int32)

    lane = jnp.arange(128, dtype=jnp.int32)[None, :]
    lab_of_lane = jnp.clip((lane - 1) // 2, 0, L - 1)
    tgt_at_lane = jnp.take_along_axis(
        targets, jnp.broadcast_to(lab_of_lane, (B, 128)), axis=1)
    cls = jnp.where(lane % 2 == 1, tgt_at_lane, 0)
    cls = cls[:, None, :]
    prev_at_lane = jnp.take_along_axis(
        targets,
        jnp.broadcast_to(jnp.clip((lane - 1) // 2 - 1, 0, L - 1), (B, 128)),
        axis=1)
    skip = (lane % 2 == 1) & (lane >= 3) & (tgt_at_lane != prev_at_lane)
    skipf = jnp.pad(skip.astype(jnp.float32),
                    ((0, 0), (0, BUFLEN - 128)))
    len16 = jnp.broadcast_to(input_lengths[:, None], (B, 16))

    w = pl.pallas_call(
        _gather_kernel,
        grid=(B,),
        in_specs=[
            pl.BlockSpec((1, T, C), lambda i: (i, 0, 0)),
            pl.BlockSpec((1, 1, 128), lambda i: (i, 0, 0)),
        ],
        out_specs=pl.BlockSpec((1, T, 128), lambda i: (i, 0, 0)),
        out_shape=jax.ShapeDtypeStruct((B, T, 128), jnp.float32),
        compiler_params=pltpu.CompilerParams(
            dimension_semantics=("arbitrary",)),
    )(log_probs, cls)

    wf = w.reshape(B, T * 128)
    m_all, e_all = _sc_dp_build(B, T)(wf, skipf, len16)

    s_last = 2 * target_lengths
    i1 = (s_last + 2)[:, None]
    i2 = (s_last + 1)[:, None]
    m1 = jnp.take_along_axis(m_all, i1, axis=1)[:, 0]
    e1 = jnp.take_along_axis(e_all, i1, axis=1)[:, 0]
    m2 = jnp.take_along_axis(m_all, i2, axis=1)[:, 0]
    e2 = jnp.take_along_axis(e_all, i2, axis=1)[:, 0]
    E = jnp.maximum(e1, e2)
    v = m1 * jnp.exp2((e1 - E).astype(jnp.float32)) + \
        m2 * jnp.exp2((e2 - E).astype(jnp.float32))
    ef = E.astype(jnp.float32)
    return -(jnp.log(v) + ef * LN2HI + ef * LN2LO)


# speedup vs baseline: 1.1764x; 1.0019x over previous
import functools

import jax
import jax.numpy as jnp
from jax import lax
from jax.experimental import pallas as pl
from jax.experimental.pallas import tpu as pltpu
from jax.experimental.pallas import tpu_sc as plsc

DEADE = -(1 << 28)
MANT_MASK = 0x007FFFFF
ONE_BITS = 0x3F800000
LN2HI = 0.69314575195
LN2LO = 1.42860677e-06

NCHUNK = 9
BUFLEN = 160


def _gather_kernel(lp_ref, cls_ref, w_ref):
    x = lp_ref[0]
    m = jnp.max(x, axis=1, keepdims=True)
    e = jnp.exp(x - m)
    z = jnp.sum(e, axis=1, keepdims=True)
    C = x.shape[1]
    cls = cls_ref[0]
    cidx = jax.lax.broadcasted_iota(jnp.int32, (C, 128), 0)
    oh = (cidx == cls).astype(jnp.float32)
    g = jnp.dot(e, oh, preferred_element_type=jnp.float32)
    w_ref[0] = g * (1.0 / z)


def _vdecomp(p):
    bits = lax.bitcast_convert_type(p, jnp.int32)
    e = lax.shift_right_logical(bits, 23) - 127
    m = lax.bitcast_convert_type(
        lax.bitwise_or(lax.bitwise_and(bits, MANT_MASK), ONE_BITS),
        jnp.float32)
    return m, e


def _vscale(d):
    return lax.bitcast_convert_type(
        lax.shift_left(jnp.maximum(d + 127, 0), 23), jnp.float32)


def _sc_dp_build(B, T):
    mesh = plsc.VectorSubcoreMesh(core_axis_name="c", subcore_axis_name="s")

    @functools.partial(
        pl.kernel,
        mesh=mesh,
        out_type=[
            jax.ShapeDtypeStruct((B, BUFLEN), jnp.float32),
            jax.ShapeDtypeStruct((B, BUFLEN), jnp.int32),
        ],
        scratch_types=[
            pltpu.VMEM((T * 128,), jnp.float32),
            pltpu.VMEM((BUFLEN,), jnp.float32),
            pltpu.VMEM((16,), jnp.int32),
            pltpu.VMEM((BUFLEN,), jnp.float32),
            pltpu.VMEM((BUFLEN,), jnp.int32),
            pltpu.VMEM((BUFLEN,), jnp.float32),
            pltpu.VMEM((BUFLEN,), jnp.int32),
        ],
    )
    def sc_dp(w_hbm, skip_hbm, len_hbm, m_out, e_out,
              w_v, skip_v, len_v, bmA, beA, bmB, beB):
        wid = lax.axis_index("s") * 2 + lax.axis_index("c")
        pltpu.sync_copy(w_hbm.at[wid], w_v)
        pltpu.sync_copy(skip_hbm.at[wid], skip_v)
        pltpu.sync_copy(len_hbm.at[wid], len_v)

        ones = jnp.full((16,), 1.0, jnp.float32)
        deade = jnp.full((16,), DEADE, jnp.int32)
        for off in range(0, BUFLEN, 16):
            bmA[pl.ds(off, 16)] = ones
            beA[pl.ds(off, 16)] = deade
            bmB[pl.ds(off, 16)] = ones
            beB[pl.ds(off, 16)] = deade
        p0 = w_v[pl.ds(0, 16)]
        m_i, e_i = _vdecomp(p0)
        lane16 = lax.broadcasted_iota(jnp.int32, (16,), 0)
        bmA[pl.ds(2, 16)] = jnp.where(lane16 < 2, m_i, 1.0)
        beA[pl.ds(2, 16)] = jnp.where(lane16 < 2, e_i, DEADE)

        lenv = len_v[...]
        skv = [skip_v[pl.ds(16 * c, 16)] for c in range(NCHUNK)]

        def step(t, src_m, src_e, dst_m, dst_e):
            act = jnp.full((16,), t, jnp.int32) < lenv
            for c in range(NCHUNK):
                base = 16 * c
                prow = min(base, 127 - 15)
                pch = w_v[pl.ds(t * 128 + prow, 16)]
                mS = src_m[pl.ds(base + 2, 16)]
                eS = src_e[pl.ds(base + 2, 16)]
                m1 = src_m[pl.ds(base + 1, 16)]
                e1 = src_e[pl.ds(base + 1, 16)]
                m2 = src_m[pl.ds(base, 16)]
                e2 = src_e[pl.ds(base, 16)]
                e2 = jnp.where(skv[c] > 0, e2, DEADE)
                E = jnp.maximum(jnp.maximum(eS, e1), e2)
                msum = (mS * _vscale(eS - E) + m1 * _vscale(e1 - E)
                        + m2 * _vscale(e2 - E)) * pch
                bits = lax.bitcast_convert_type(msum, jnp.int32)
                eb = lax.shift_right_logical(bits, 23)
                mN = lax.bitcast_convert_type(
                    lax.bitwise_or(lax.bitwise_and(bits, MANT_MASK),
                                   ONE_BITS), jnp.float32)
                eN = E + (eb - 127)
                dst_m[pl.ds(base + 2, 16)] = jnp.where(act, mN, mS)
                dst_e[pl.ds(base + 2, 16)] = jnp.where(act, eN, eS)

        def pair(k, carry):
            step(1 + 2 * k, bmA, beA, bmB, beB)
            step(2 + 2 * k, bmB, beB, bmA, beA)
            return carry

        lax.fori_loop(0, (T - 2) // 2, pair, 0)
        step(T - 1, bmA, beA, bmB, beB)

        pltpu.sync_copy(bmB, m_out.at[wid])
        pltpu.sync_copy(beB, e_out.at[wid])

    return sc_dp


@jax.jit
def kernel(log_probs, targets, input_lengths, target_lengths):
    B, T, C = log_probs.shape
    L = targets.shape[1]
    targets = targets.astype(jnp.int32)
    input_lengths = input_lengths.astype(jnp.int32)
    target_lengths = target_lengths.astype(jnp.int32)

    lane = jnp.arange(128, dtype=jnp.int32)[None, :]
    lab_of_lane = jnp.clip((lane - 1) // 2, 0, L - 1)
    tgt_at_lane = jnp.take_along_axis(
        targets, jnp.broadcast_to(lab_of_lane, (B, 128)), axis=1)
    cls = jnp.where(lane % 2 == 1, tgt_at_lane, 0)
    cls = cls[:, None, :]
    prev_at_lane = jnp.take_along_axis(
        targets,
        jnp.broadcast_to(jnp.clip((lane - 1) // 2 - 1, 0, L - 1), (B, 128)),
        axis=1)
    skip = (lane % 2 == 1) & (lane >= 3) & (tgt_at_lane != prev_at_lane)
    skipf = jnp.pad(skip.astype(jnp.float32),
                    ((0, 0), (0, BUFLEN - 128)))
    len16 = jnp.broadcast_to(input_lengths[:, None], (B, 16))

    w = pl.pallas_call(
        _gather_kernel,
        grid=(B,),
        in_specs=[
            pl.BlockSpec((1, T, C), lambda i: (i, 0, 0)),
            pl.BlockSpec((1, 1, 128), lambda i: (i, 0, 0)),
        ],
        out_specs=pl.BlockSpec((1, T, 128), lambda i: (i, 0, 0)),
        out_shape=jax.ShapeDtypeStruct((B, T, 128), jnp.float32),
        compiler_params=pltpu.CompilerParams(
            dimension_semantics=("arbitrary",)),
    )(log_probs, cls)

    wf = w.reshape(B, T * 128)
    m_all, e_all = _sc_dp_build(B, T)(wf, skipf, len16)

    s_last = 2 * target_lengths
    i1 = (s_last + 2)[:, None]
    i2 = (s_last + 1)[:, None]
    m1 = jnp.take_along_axis(m_all, i1, axis=1)[:, 0]
    e1 = jnp.take_along_axis(e_all, i1, axis=1)[:, 0]
    m2 = jnp.take_along_axis(m_all, i2, axis=1)[:, 0]
    e2 = jnp.take_along_axis(e_all, i2, axis=1)[:, 0]
    E = jnp.maximum(e1, e2)
    v = m1 * jnp.exp2((e1 - E).astype(jnp.float32)) + \
        m2 * jnp.exp2((e2 - E).astype(jnp.float32))
    ef = E.astype(jnp.float32)
    return -(jnp.log(v) + ef * LN2HI + ef * LN2LO)


# speedup vs baseline: 1.8244x; 1.5508x over previous
import functools

import jax
import jax.numpy as jnp
from jax import lax
from jax.experimental import pallas as pl
from jax.experimental.pallas import tpu as pltpu
from jax.experimental.pallas import tpu_sc as plsc

DEADE = -(1 << 28)
MANT_MASK = 0x007FFFFF
ONE_BITS = 0x3F800000
LN2HI = 0.69314575195
LN2LO = 1.42860677e-06

NCHUNK = 9
BUFLEN = 160


def _gather_kernel(lp_ref, cls_ref, w_ref):
    x = lp_ref[0]
    m = jnp.max(x, axis=1, keepdims=True)
    e = jnp.exp(x - m)
    z = jnp.sum(e, axis=1, keepdims=True)
    C = x.shape[1]
    cls = cls_ref[0]
    cidx = jax.lax.broadcasted_iota(jnp.int32, (C, 128), 0)
    oh = (cidx == cls).astype(jnp.float32)
    g = jnp.dot(e, oh, preferred_element_type=jnp.float32)
    w_ref[0] = g * (1.0 / z)


def _vdecomp(p):
    bits = lax.bitcast_convert_type(p, jnp.int32)
    e = lax.shift_right_logical(bits, 23) - 127
    m = lax.bitcast_convert_type(
        lax.bitwise_or(lax.bitwise_and(bits, MANT_MASK), ONE_BITS),
        jnp.float32)
    return m, e


def _vscale(d):
    return lax.bitcast_convert_type(
        lax.shift_left(jnp.maximum(d + 127, 0), 23), jnp.float32)


def _sc_dp_build(B, T):
    mesh = plsc.VectorSubcoreMesh(core_axis_name="c", subcore_axis_name="s")

    @functools.partial(
        pl.kernel,
        mesh=mesh,
        out_type=[
            jax.ShapeDtypeStruct((B, BUFLEN), jnp.float32),
            jax.ShapeDtypeStruct((B, BUFLEN), jnp.int32),
        ],
        scratch_types=[
            pltpu.VMEM((T * 128,), jnp.float32),
            pltpu.VMEM((BUFLEN,), jnp.float32),
            pltpu.VMEM((16,), jnp.int32),
            pltpu.VMEM((BUFLEN,), jnp.float32),
            pltpu.VMEM((BUFLEN,), jnp.int32),
            pltpu.VMEM((BUFLEN,), jnp.float32),
            pltpu.VMEM((BUFLEN,), jnp.int32),
        ],
    )
    def sc_dp(w_hbm, skip_hbm, len_hbm, m_out, e_out,
              w_v, skip_v, len_v, bmA, beA, bmB, beB):
        wid = lax.axis_index("s") * 2 + lax.axis_index("c")
        pltpu.sync_copy(w_hbm.at[wid], w_v)
        pltpu.sync_copy(skip_hbm.at[wid], skip_v)
        pltpu.sync_copy(len_hbm.at[wid], len_v)

        ones = jnp.full((16,), 1.0, jnp.float32)
        deade = jnp.full((16,), DEADE, jnp.int32)
        for off in range(0, BUFLEN, 16):
            bmA[pl.ds(off, 16)] = ones
            beA[pl.ds(off, 16)] = deade
            bmB[pl.ds(off, 16)] = ones
            beB[pl.ds(off, 16)] = deade
        p0 = w_v[pl.ds(0, 16)]
        m_i, e_i = _vdecomp(p0)
        lane16 = lax.broadcasted_iota(jnp.int32, (16,), 0)
        bmA[pl.ds(2, 16)] = jnp.where(lane16 < 2, m_i, 1.0)
        beA[pl.ds(2, 16)] = jnp.where(lane16 < 2, e_i, DEADE)

        lenv = len_v[...]
        skv = [skip_v[pl.ds(16 * c, 16)] for c in range(NCHUNK)]

        def step(t, src_m, src_e, dst_m, dst_e):
            act = jnp.full((16,), t, jnp.int32) < lenv
            for c in range(NCHUNK):
                base = 16 * c
                prow = min(base, 127 - 15)
                pch = w_v[pl.ds(t * 128 + prow, 16)]
                mS = src_m[pl.ds(base + 2, 16)]
                eS = src_e[pl.ds(base + 2, 16)]
                m1 = src_m[pl.ds(base + 1, 16)]
                e1 = src_e[pl.ds(base + 1, 16)]
                m2 = src_m[pl.ds(base, 16)]
                e2 = src_e[pl.ds(base, 16)]
                e2 = jnp.where(skv[c] > 0, e2, DEADE)
                E = jnp.maximum(jnp.maximum(eS, e1), e2)
                msum = (mS * _vscale(eS - E) + m1 * _vscale(e1 - E)
                        + m2 * _vscale(e2 - E)) * pch
                bits = lax.bitcast_convert_type(msum, jnp.int32)
                eb = lax.shift_right_logical(bits, 23)
                mN = lax.bitcast_convert_type(
                    lax.bitwise_or(lax.bitwise_and(bits, MANT_MASK),
                                   ONE_BITS), jnp.float32)
                eN = E + (eb - 127)
                dst_m[pl.ds(base + 2, 16)] = jnp.where(act, mN, mS)
                dst_e[pl.ds(base + 2, 16)] = jnp.where(act, eN, eS)

        def pair(k, carry):
            step(1 + 2 * k, bmA, beA, bmB, beB)
            step(2 + 2 * k, bmB, beB, bmA, beA)
            return carry

        lax.fori_loop(0, (T - 2) // 2, pair, 0)
        step(T - 1, bmA, beA, bmB, beB)

        pltpu.sync_copy(bmB, m_out.at[wid])
        pltpu.sync_copy(beB, e_out.at[wid])

    return sc_dp


@jax.jit
def kernel(log_probs, targets, input_lengths, target_lengths):
    B, T, C = log_probs.shape
    L = targets.shape[1]
    targets = targets.astype(jnp.int32)
    input_lengths = input_lengths.astype(jnp.int32)
    target_lengths = target_lengths.astype(jnp.int32)

    lane = jnp.arange(128, dtype=jnp.int32)[None, :]
    j64 = jnp.arange(L, dtype=jnp.int32)[:, None]
    lab = (lane - 1) // 2
    is_lab = lane % 2 == 1
    sel_cur = (is_lab & (lab == j64)).astype(jnp.float32)
    sel_prv = (is_lab & (lane >= 3) & (lab - 1 == j64)).astype(jnp.float32)
    tgtf = targets.astype(jnp.float32)
    tat = tgtf @ sel_cur
    pat = tgtf @ sel_prv
    cls = tat.astype(jnp.int32)[:, None, :]
    skip = is_lab & (lane >= 3) & (tat != pat)
    skipf = jnp.pad(skip.astype(jnp.float32),
                    ((0, 0), (0, BUFLEN - 128)))
    len16 = jnp.broadcast_to(input_lengths[:, None], (B, 16))

    w = pl.pallas_call(
        _gather_kernel,
        grid=(B,),
        in_specs=[
            pl.BlockSpec((1, T, C), lambda i: (i, 0, 0)),
            pl.BlockSpec((1, 1, 128), lambda i: (i, 0, 0)),
        ],
        out_specs=pl.BlockSpec((1, T, 128), lambda i: (i, 0, 0)),
        out_shape=jax.ShapeDtypeStruct((B, T, 128), jnp.float32),
        compiler_params=pltpu.CompilerParams(
            dimension_semantics=("arbitrary",)),
    )(log_probs, cls)

    wf = w.reshape(B, T * 128)
    m_all, e_all = _sc_dp_build(B, T)(wf, skipf, len16)

    s_last = 2 * target_lengths
    lane_b = jnp.arange(BUFLEN, dtype=jnp.int32)[None, :]
    sel1 = (lane_b == (s_last + 2)[:, None]).astype(jnp.float32)
    sel2 = (lane_b == (s_last + 1)[:, None]).astype(jnp.float32)
    eaf = e_all.astype(jnp.float32)
    m1 = jnp.sum(m_all * sel1, axis=1)
    e1 = jnp.sum(eaf * sel1, axis=1)
    m2 = jnp.sum(m_all * sel2, axis=1)
    e2 = jnp.sum(eaf * sel2, axis=1)
    E = jnp.maximum(e1, e2)
    v = m1 * jnp.exp2((e1 - E).astype(jnp.float32)) + \
        m2 * jnp.exp2((e2 - E).astype(jnp.float32))
    ef = E.astype(jnp.float32)
    return -(jnp.log(v) + ef * LN2HI + ef * LN2LO)
